# Initial kernel scaffold; baseline (speedup 1.0000x reference)
import jax
import jax.numpy as jnp
from jax.experimental import pallas as pl


def kernel(vectors, node_specie, senders, receivers, W_embed, W_up0, Wr1_0, Wr2_0, Wc0, Wlin0, Wro0, W_up1, Wr1_1, Wr2_1, Wc1, Wsc_lin1, Wsc_sp1, Wlin1, Wro1a, Wro1b):
    raise NotImplementedError("write your pallas kernel here")


# speedup vs baseline: 1.9524x; 1.9524x over previous
import jax
import jax.numpy as jnp
import numpy as np
from jax.experimental import pallas as pl

N = 10000
E = 160000
NUM_SPECIES = 10
F = 128
NB = 8
SH = 9
R_MAX = 5.0
EPS = 0.5


def _sph(u):
    x, y, z = u[:, 0], u[:, 1], u[:, 2]
    s3 = float(np.sqrt(3.0)); s15 = float(np.sqrt(15.0)); s5 = float(np.sqrt(5.0))
    comps = [jnp.ones_like(x), s3 * x, s3 * y, s3 * z,
             s15 * x * y, s15 * y * z, 0.5 * s5 * (3.0 * z * z - 1.0),
             s15 * x * z, 0.5 * s15 * (x * x - y * y)]
    return jnp.stack(comps, axis=-1)


def _radial(r):
    n = jnp.arange(1, NB + 1, dtype=jnp.float32)
    rs = jnp.clip(r, 1e-9, None)
    rb = np.sqrt(2.0 / R_MAX) * jnp.sin(n * jnp.pi * rs / R_MAX) / rs
    x = r / R_MAX
    env = 1.0 - 21.0 * x ** 5 + 35.0 * x ** 6 - 15.0 * x ** 7
    env = jnp.where(x < 1.0, env, 0.0)
    return rb * env


def kernel(vectors, node_specie, senders, receivers, W_embed, W_up0, Wr1_0, Wr2_0, Wc0, Wlin0, Wro0, W_up1, Wr1_1, Wr2_1, Wc1, Wsc_lin1, Wsc_sp1, Wlin1, Wro1a, Wro1b):
    lengths = jnp.sqrt(jnp.sum(vectors * vectors, axis=-1, keepdims=True) + 1e-12)
    Y = _sph(vectors / lengths)
    ef = _radial(lengths)
    R0 = jax.nn.silu(ef @ Wr1_0) @ Wr2_0
    R1 = jax.nn.silu(ef @ Wr1_1) @ Wr2_1
    yr0 = Y * R0
    yr1 = Y * R1

    spec_send = node_specie[senders]
    C0 = jnp.zeros((N * NUM_SPECIES, SH), jnp.float32).at[
        receivers * NUM_SPECIES + spec_send].add(yr0).reshape(N, NUM_SPECIES, SH)
    T0 = W_embed @ W_up0
    A0 = jnp.einsum('nsa,sf->naf', C0, T0, precision='highest') * EPS
    scal0 = jnp.sum(A0 * A0, axis=1)
    cw = Wc0[node_specie]
    g1 = (A0[:, 0, :] * (cw[:, 0] + cw[:, 1] * scal0 + cw[:, 2] * scal0 * scal0)) @ Wlin0
    ro0 = g1 @ Wro0

    h1 = g1 @ W_up1
    A1 = jnp.zeros((N, SH, F), jnp.float32).at[receivers].add(
        yr1[:, :, None] * h1[senders][:, None, :]) * EPS
    scal1 = jnp.sum(A1 * A1, axis=1)
    cw1 = Wc1[node_specie]
    nf2_0 = (A1[:, 0, :] * (cw1[:, 0] + cw1[:, 1] * scal1 + cw1[:, 2] * scal1 * scal1)) @ Wlin1 \
        + (g1 @ Wsc_lin1) * Wsc_sp1[node_specie]
    ro1 = jax.nn.silu(nf2_0 @ Wro1a) @ Wro1b
    return jnp.stack([ro0, ro1], axis=1)


# speedup vs baseline: 21.1855x; 10.8508x over previous
import functools

import jax
import jax.numpy as jnp
import numpy as np
from jax import lax
from jax.experimental import pallas as pl
from jax.experimental.pallas import tpu as pltpu
from jax.experimental.pallas import tpu_sc as plsc

N = 10000
E = 160000
NS_ = 10
F = 128
NB = 8
SH = 9
R_MAX = 5.0
EPS = 0.5
HR = 64
HRO = 16

NP_ = 10240
BN = 512
BE = 3200
EB = 3200
NW = 32
KNODE = NP_ // NW
CAP = 6128
GB = 192
GBC = 512
FC = 32


def _k1_body(v_ref, w10_ref, w20_ref, w11_ref, w21_ref, yr0_ref, yr1_ref):
    x = v_ref[0:1, :]
    y = v_ref[1:2, :]
    z = v_ref[2:3, :]
    ll = jnp.sqrt(x * x + y * y + z * z + 1e-12)
    ux, uy, uz = x / ll, y / ll, z / ll
    s3 = float(np.sqrt(3.0)); s15 = float(np.sqrt(15.0)); s5 = float(np.sqrt(5.0))
    Y = jnp.concatenate([
        jnp.ones_like(ux), s3 * ux, s3 * uy, s3 * uz,
        s15 * ux * uy, s15 * uy * uz, 0.5 * s5 * (3.0 * uz * uz - 1.0),
        s15 * ux * uz, 0.5 * s15 * (ux * ux - uy * uy)], axis=0)

    rs = jnp.clip(ll, 1e-9, None)
    rows = [np.sqrt(2.0 / R_MAX) * jnp.sin((float(n) * np.pi / R_MAX) * rs) / rs
            for n in range(1, NB + 1)]
    xq = ll / R_MAX
    env = 1.0 - 21.0 * xq ** 5 + 35.0 * xq ** 6 - 15.0 * xq ** 7
    env = jnp.where(xq < 1.0, env, 0.0)
    ef = jnp.concatenate(rows, axis=0) * env

    def head(w1t, w2t):
        h = jax.nn.silu(jnp.dot(w1t, ef, preferred_element_type=jnp.float32))
        return jnp.dot(w2t, h, preferred_element_type=jnp.float32)

    z7 = jnp.zeros((16 - SH, BE), jnp.float32)
    yr0_ref[...] = jnp.concatenate(
        [Y * head(w10_ref[...], w20_ref[...]), z7], axis=0).T
    yr1_ref[...] = jnp.concatenate(
        [Y * head(w11_ref[...], w21_ref[...]), z7], axis=0).T


def _edge_geometry(vectors, Wr1_0, Wr2_0, Wr1_1, Wr2_1):
    vT = vectors.T
    wfull = lambda shp: pl.BlockSpec(shp, lambda i: (0, 0))
    yr0T, yr1T = pl.pallas_call(
        _k1_body,
        grid=(E // BE,),
        in_specs=[pl.BlockSpec((3, BE), lambda i: (0, i)),
                  wfull((HR, NB)), wfull((SH, HR)),
                  wfull((HR, NB)), wfull((SH, HR))],
        out_specs=[pl.BlockSpec((BE, 16), lambda i: (i, 0)),
                   pl.BlockSpec((BE, 16), lambda i: (i, 0))],
        out_shape=[jax.ShapeDtypeStruct((E, 16), jnp.float32),
                   jax.ShapeDtypeStruct((E, 16), jnp.float32)],
    )(vT, Wr1_0.T, Wr2_0.T, Wr1_1.T, Wr2_1.T)
    return yr0T, yr1T


def _c0_body(send_h, recv_h, yr0_h, nspec_h, zt_h, c0_h, cs_h, cw_h, cnt_h,
             ns_v, sb0, sb1, rb0, rb1, cs_v, cw_v, eb0, eb1, yg0, yg1,
             tab_v, cb_v, sem0, sem1):
    cid = lax.axis_index("c")
    sid = lax.axis_index("s")
    wid = sid * 2 + cid
    base = wid * KNODE
    iota = lax.iota(jnp.int32, 16)
    zi = jnp.zeros((16,), jnp.int32)
    sbuf = (sb0, sb1)
    rbuf = (rb0, rb1)
    ebuf = (eb0, eb1)
    ygb = (yg0, yg1)
    sems = (sem0, sem1)

    def zf(i, _):
        cs_v[pl.ds(i * 16, 16)] = zi
        cw_v[pl.ds(i * 16, 16)] = zi
        return 0
    lax.fori_loop(0, (CAP + 16) // 16, zf, 0)

    pltpu.sync_copy(nspec_h, ns_v)

    NSB = E // EB

    def scan_issue(b, u):
        pltpu.async_copy(send_h.at[pl.ds(b * EB, EB)], sbuf[u], sems[u])
        pltpu.async_copy(recv_h.at[pl.ds(b * EB, EB)], rbuf[u], sems[u])

    def scan_drain(u):
        pltpu.make_async_copy(send_h.at[pl.ds(0, EB)], sbuf[u], sems[u]).wait()
        pltpu.make_async_copy(recv_h.at[pl.ds(0, EB)], rbuf[u], sems[u]).wait()

    @pl.when(sid >= 0)
    def _():
        scan_issue(0, 0)

    def scan_body2(b, cnt, u):
        scan_drain(u)

        @pl.when(b + 1 < NSB)
        def _():
            scan_issue(b + 1, 1 - u)
        off = b * EB

        def j_body(j, cnt):
            rv = rbuf[u][pl.ds(j * 16, 16)]
            sv = sbuf[u][pl.ds(j * 16, 16)]
            spec = plsc.load_gather(ns_v, [sv])
            lv = rv - base
            m = (lv >= 0) & (lv < KNODE)
            eid = jnp.full((16,), off + j * 16, jnp.int32) + iota
            pw = jnp.left_shift(spec, 27) + jnp.left_shift(lv, 18) + eid

            @pl.when(cnt <= CAP - 16)
            def _():
                plsc.store_compressed(cs_v.at[pl.ds(cnt, 16)], sv, mask=m)
                plsc.store_compressed(cw_v.at[pl.ds(cnt, 16)], pw, mask=m)
            return cnt + jnp.sum(m.astype(jnp.int32))
        return lax.fori_loop(0, EB // 16, j_body, cnt)

    def scan_pair(p, cnt):
        cnt = scan_body2(p * 2, cnt, 0)
        return scan_body2(p * 2 + 1, cnt, 1)

    cnt = lax.fori_loop(0, NSB // 2, scan_pair, 0)

    pltpu.sync_copy(zt_h, tab_v)
    NAB = (CAP + GBC - 1) // GBC

    def acc_issue(gb, u):
        def eid_extract(j, _):
            v = cw_v[pl.ds(gb * GBC + j * 16, 16)]
            ebuf[u][pl.ds(j * 16, 16)] = v & 262143
            return 0
        lax.fori_loop(0, GBC // 16, eid_extract, 0)
        pltpu.async_copy(yr0_h.at[ebuf[u]], ygb[u], sems[u])

    @pl.when(sid >= 0)
    def _():
        acc_issue(0, 0)

    def acc_body(gb, _, u):
        pltpu.make_async_copy(yr0_h.at[pl.ds(0, GBC)], ygb[u], sems[u]).wait()

        @pl.when(gb + 1 < NAB)
        def _():
            acc_issue(gb + 1, 1 - u)

        def e_body(e, _):
            pk = cw_v[pl.ds(gb * GBC + e, 16)][0]
            spec = jnp.right_shift(pk, 27)
            local = jnp.bitwise_and(jnp.right_shift(pk, 18), 511)
            yrow = ygb[u][e, pl.ds(0, 16)]
            idxv = iota + (local * NS_ + spec) * 16
            plsc.addupdate_scatter(tab_v, [idxv], yrow)
            return 0
        nleft = jnp.maximum(jnp.minimum(cnt - gb * GBC, GBC), 0)
        lax.fori_loop(0, nleft, e_body, 0)
        return 0

    def acc_pair(p, _):
        acc_body(p * 2, 0, 0)
        acc_body(p * 2 + 1, 0, 1)
        return 0

    lax.fori_loop(0, NAB // 2, acc_pair, 0)

    pltpu.sync_copy(tab_v, c0_h.at[pl.ds(wid * KNODE * NS_ * 16,
                                         KNODE * NS_ * 16)])
    pltpu.sync_copy(cs_v, cs_h.at[pl.ds(wid * (CAP + 16), CAP + 16)])
    pltpu.sync_copy(cw_v, cw_h.at[pl.ds(wid * (CAP + 16), CAP + 16)])
    cb_v[pl.ds(0, 16)] = jnp.full((16,), cnt, jnp.int32)
    pltpu.sync_copy(cb_v, cnt_h.at[pl.ds(wid * 16, 16)])


def _c0_scatter(senders, receivers, yr0, node_specie):
    mesh = plsc.VectorSubcoreMesh(core_axis_name="c", subcore_axis_name="s",
                                  num_cores=2)
    k = functools.partial(
        pl.kernel, mesh=mesh,
        compiler_params=pltpu.CompilerParams(needs_layout_passes=False,
                                            use_tc_tiling_on_sc=False),
        out_type=[jax.ShapeDtypeStruct((NP_ * NS_ * 16,), jnp.float32),
                  jax.ShapeDtypeStruct((NW * (CAP + 16),), jnp.int32),
                  jax.ShapeDtypeStruct((NW * (CAP + 16),), jnp.int32),
                  jax.ShapeDtypeStruct((NW * 16,), jnp.int32)],
        scratch_types=[
            pltpu.VMEM((N,), jnp.int32),
            pltpu.VMEM((EB,), jnp.int32),
            pltpu.VMEM((EB,), jnp.int32),
            pltpu.VMEM((EB,), jnp.int32),
            pltpu.VMEM((EB,), jnp.int32),
            pltpu.VMEM((CAP + 16,), jnp.int32),
            pltpu.VMEM((CAP + 16,), jnp.int32),
            pltpu.VMEM((GBC,), jnp.int32),
            pltpu.VMEM((GBC,), jnp.int32),
            pltpu.VMEM((GBC, 16), jnp.float32),
            pltpu.VMEM((GBC, 16), jnp.float32),
            pltpu.VMEM((KNODE * NS_ * 16,), jnp.float32),
            pltpu.VMEM((16,), jnp.int32),
            pltpu.SemaphoreType.DMA,
            pltpu.SemaphoreType.DMA,
        ])(_c0_body)
    zt = jnp.zeros((KNODE * NS_ * 16,), jnp.float32)
    return k(senders, receivers, yr0, node_specie, zt)


def _k3_body(c0_ref, oh_ref, t0_ref, wc_ref, wlin_ref, wup_ref,
             wsc_ref, wsp_ref, wro_ref, h1_ref, gsc_ref, ro0_ref):
    cc = c0_ref[...]
    acc = jnp.zeros((BN, 16, F), jnp.float32)
    for s in range(NS_):
        acc = acc + cc[:, s, :][:, :, None] * t0_ref[s, :][None, None, :]
    A = acc * EPS
    scal = jnp.sum(A * A, axis=1)
    a0 = A[:, 0, :]

    oh = oh_ref[...]
    cw0 = jnp.zeros((BN, F), jnp.float32)
    cw1 = jnp.zeros((BN, F), jnp.float32)
    cw2 = jnp.zeros((BN, F), jnp.float32)
    wsp = jnp.zeros((BN, F), jnp.float32)
    for s in range(NS_):
        o = oh[:, s][:, None]
        cw0 = cw0 + o * wc_ref[s, 0, :][None, :]
        cw1 = cw1 + o * wc_ref[s, 1, :][None, :]
        cw2 = cw2 + o * wc_ref[s, 2, :][None, :]
        wsp = wsp + o * wsp_ref[s, :][None, :]

    b0 = cw0 * a0 + (cw1 * a0) * scal + (cw2 * a0) * (scal * scal)
    g1 = jnp.dot(b0, wlin_ref[...], preferred_element_type=jnp.float32)
    h1_ref[...] = jnp.dot(g1, wup_ref[...], preferred_element_type=jnp.float32)
    gsc_ref[...] = jnp.dot(g1, wsc_ref[...],
                           preferred_element_type=jnp.float32) * wsp
    ro0_ref[...] = jnp.dot(g1, wro_ref[...], preferred_element_type=jnp.float32)


def _node_stage0(c0, onehot, T0, Wc0, Wlin0, W_up1, Wsc_lin1, Wsc_sp1,
                 Wro0p):
    wfull = lambda *shp: pl.BlockSpec(shp, lambda i: (0,) * len(shp))
    return pl.pallas_call(
        _k3_body,
        grid=(NP_ // BN,),
        in_specs=[pl.BlockSpec((BN, NS_, 16), lambda i: (i, 0, 0)),
                  pl.BlockSpec((BN, NS_), lambda i: (i, 0)),
                  wfull(NS_, F), wfull(NS_, 3, F), wfull(F, F), wfull(F, F),
                  wfull(F, F), wfull(NS_, F), wfull(F, 8)],
        out_specs=[pl.BlockSpec((BN, F), lambda i: (i, 0)),
                   pl.BlockSpec((BN, F), lambda i: (i, 0)),
                   pl.BlockSpec((BN, 8), lambda i: (i, 0))],
        out_shape=[jax.ShapeDtypeStruct((NP_, F), jnp.float32),
                   jax.ShapeDtypeStruct((NP_, F), jnp.float32),
                   jax.ShapeDtypeStruct((NP_, 8), jnp.float32)],
    )(c0, onehot, T0, Wc0, Wlin0, W_up1, Wsc_lin1, Wsc_sp1, Wro0p)


def _a1_body(yr1_h, h0_h, h1_h, h2_h, h3_h, cs_h, cw_h, cnt_h, zt_h, a1_h,
             cs_v, cw_v, eb0, eb1, hg0, hg1, yg0, yg1,
             tab_v, cb_v, sem0, sem1):
    cid = lax.axis_index("c")
    sid = lax.axis_index("s")
    wid = sid * 2 + cid
    base = wid * KNODE
    iota = lax.iota(jnp.int32, 16)
    ebuf = (eb0, eb1)
    hgb = (hg0, hg1)
    ygb = (yg0, yg1)
    sems = (sem0, sem1)
    lbase = wid * (CAP + 16)

    pltpu.sync_copy(cs_h.at[pl.ds(lbase, CAP + 16)], cs_v)
    pltpu.sync_copy(cw_h.at[pl.ds(lbase, CAP + 16)], cw_v)
    pltpu.sync_copy(cnt_h.at[pl.ds(wid * 16, 16)], cb_v)
    cnt = cb_v[pl.ds(0, 16)][0]
    NBLK = (CAP + 16) // GB

    for c, h_h in enumerate((h0_h, h1_h, h2_h, h3_h)):
        pltpu.sync_copy(zt_h, tab_v)

        def issue(gb, u, h_h=h_h):
            def eid_extract(j, _):
                v = cw_v[pl.ds(gb * GB + j * 16, 16)]
                ebuf[u][pl.ds(j * 16, 16)] = v & 262143
                return 0
            lax.fori_loop(0, GB // 16, eid_extract, 0)
            pltpu.async_copy(h_h.at[cs_v.at[pl.ds(gb * GB, GB)]], hgb[u],
                             sems[u])
            pltpu.async_copy(yr1_h.at[ebuf[u]], ygb[u], sems[u])

        issue(0, 0)

        def blk_body(gb, _, u, h_h=h_h):
            @pl.when(gb * GB < cnt)
            def _():
                pltpu.make_async_copy(h_h.at[pl.ds(0, GB)], hgb[u],
                                      sems[u]).wait()
                pltpu.make_async_copy(yr1_h.at[pl.ds(0, GB)], ygb[u],
                                      sems[u]).wait()

                @pl.when((gb + 1) * GB < cnt)
                def _():
                    issue(gb + 1, 1 - u)

                def e_body(e, _):
                    pk = cw_v[pl.ds(gb * GB + e, 16)][0]
                    local = jnp.bitwise_and(jnp.right_shift(pk, 18), 511)
                    hrow0 = hgb[u][e, pl.ds(0, 16)]
                    hrow1 = hgb[u][e, pl.ds(16, 16)]
                    yrow = ygb[u][e, pl.ds(0, 16)]
                    rowbase = local * (SH * FC) + iota
                    for a in range(SH):
                        yv = yrow[a]
                        idxv = rowbase + a * FC
                        plsc.addupdate_scatter(tab_v, [idxv], hrow0 * yv)
                        plsc.addupdate_scatter(tab_v, [idxv + 16], hrow1 * yv)
                    return 0
                nleft = jnp.minimum(cnt - gb * GB, GB)
                lax.fori_loop(0, nleft, e_body, 0)
            return 0

        def blk_pair(p, _):
            blk_body(p * 2, 0, 0)
            blk_body(p * 2 + 1, 0, 1)
            return 0

        lax.fori_loop(0, NBLK // 2, blk_pair, 0)

        pltpu.sync_copy(tab_v,
                        a1_h.at[c, pl.ds(base * SH * FC, KNODE * SH * FC)])


def _a1_scatter(yr1, h1c, cs, cw, cnt):
    mesh = plsc.VectorSubcoreMesh(core_axis_name="c", subcore_axis_name="s",
                                  num_cores=2)
    k = functools.partial(
        pl.kernel, mesh=mesh,
        compiler_params=pltpu.CompilerParams(needs_layout_passes=False,
                                            use_tc_tiling_on_sc=False),
        out_type=jax.ShapeDtypeStruct((4, NP_ * SH * FC), jnp.float32),
        scratch_types=[
            pltpu.VMEM((CAP + 16,), jnp.int32),
            pltpu.VMEM((CAP + 16,), jnp.int32),
            pltpu.VMEM((GB,), jnp.int32),
            pltpu.VMEM((GB,), jnp.int32),
            pltpu.VMEM((GB, FC), jnp.float32),
            pltpu.VMEM((GB, FC), jnp.float32),
            pltpu.VMEM((GB, 16), jnp.float32),
            pltpu.VMEM((GB, 16), jnp.float32),
            pltpu.VMEM((KNODE * SH * FC,), jnp.float32),
            pltpu.VMEM((16,), jnp.int32),
            pltpu.SemaphoreType.DMA,
            pltpu.SemaphoreType.DMA,
        ])(_a1_body)
    zt = jnp.zeros((KNODE * SH * FC,), jnp.float32)
    out = k(yr1, h1c[0], h1c[1], h1c[2], h1c[3], cs, cw, cnt, zt)
    return out.reshape(4, NP_ * SH, FC)


def _k5_body(a1_ref, oh_ref, gsc_ref, wc_ref, wlin_ref, wa_ref, wb_ref,
             ro1_ref):
    A = jnp.concatenate(
        [a1_ref[c].reshape(BN, SH, FC) for c in range(4)], axis=2) * EPS
    scal = jnp.sum(A * A, axis=1)
    a10 = A[:, 0, :]

    oh = oh_ref[...]
    cw0 = jnp.zeros((BN, F), jnp.float32)
    cw1 = jnp.zeros((BN, F), jnp.float32)
    cw2 = jnp.zeros((BN, F), jnp.float32)
    for s in range(NS_):
        o = oh[:, s][:, None]
        cw0 = cw0 + o * wc_ref[s, 0, :][None, :]
        cw1 = cw1 + o * wc_ref[s, 1, :][None, :]
        cw2 = cw2 + o * wc_ref[s, 2, :][None, :]

    b1 = cw0 * a10 + (cw1 * a10) * scal + (cw2 * a10) * (scal * scal)
    nf2 = jnp.dot(b1, wlin_ref[...], preferred_element_type=jnp.float32) \
        + gsc_ref[...]
    hro = jax.nn.silu(jnp.dot(nf2, wa_ref[...],
                              preferred_element_type=jnp.float32))
    ro1_ref[...] = jnp.dot(hro, wb_ref[...], preferred_element_type=jnp.float32)


def _node_stage1(a1, onehot, gsc, Wc1, Wlin1, Wro1a, Wro1bp):
    wfull = lambda *shp: pl.BlockSpec(shp, lambda i: (0,) * len(shp))
    return pl.pallas_call(
        _k5_body,
        grid=(NP_ // BN,),
        in_specs=[pl.BlockSpec((4, BN * SH, FC), lambda i: (0, i, 0)),
                  pl.BlockSpec((BN, NS_), lambda i: (i, 0)),
                  pl.BlockSpec((BN, F), lambda i: (i, 0)),
                  wfull(NS_, 3, F), wfull(F, F), wfull(F, HRO), wfull(HRO, 8)],
        out_specs=[pl.BlockSpec((BN, 8), lambda i: (i, 0))],
        out_shape=[jax.ShapeDtypeStruct((NP_, 8), jnp.float32)],
    )(a1, onehot, gsc, Wc1, Wlin1, Wro1a, Wro1bp)[0]


def kernel(vectors, node_specie, senders, receivers, W_embed, W_up0, Wr1_0,
           Wr2_0, Wc0, Wlin0, Wro0, W_up1, Wr1_1, Wr2_1, Wc1, Wsc_lin1,
           Wsc_sp1, Wlin1, Wro1a, Wro1b):
    yr0, yr1 = _edge_geometry(vectors, Wr1_0, Wr2_0, Wr1_1, Wr2_1)

    c0, cs, cw, cnt = _c0_scatter(senders, receivers, yr0, node_specie)
    c0 = c0.reshape(NP_, NS_, 16)

    ns_pad = jnp.concatenate([node_specie,
                              jnp.zeros((NP_ - N,), node_specie.dtype)])
    onehot = (ns_pad[:, None] == jnp.arange(NS_)[None, :]).astype(jnp.float32)

    T0 = W_embed @ W_up0
    Wro0p = jnp.pad(Wro0, ((0, 0), (0, 7)))
    h1, gsc, ro0p = _node_stage0(c0, onehot, T0, Wc0, Wlin0, W_up1,
                                 Wsc_lin1, Wsc_sp1, Wro0p)

    h1c = h1.reshape(NP_, 4, FC).transpose(1, 0, 2)
    h1c = [h1c[i] for i in range(4)]
    a1 = _a1_scatter(yr1, h1c, cs, cw, cnt)

    Wro1bp = jnp.pad(Wro1b, ((0, 0), (0, 7)))
    ro1p = _node_stage1(a1, onehot, gsc, Wc1, Wlin1, Wro1a, Wro1bp)

    return jnp.stack([ro0p[:N, 0:1], ro1p[:N, 0:1]], axis=1)
